# baseline (speedup 1.0000x reference)
import functools
import numpy as np
import jax
import jax.numpy as jnp
from jax import lax
from jax.experimental import pallas as pl
from jax.experimental.pallas import tpu as pltpu
from jax.experimental.pallas import tpu_sc as plsc

_NWORDS = 8192
_K = 50
_EMB = 64
_NC, _NS = 2, 16
_NWK = _NC * _NS
_B = _NWORDS * _K
_BPW = _B // _NWK
_PPW = _BPW // 2
_CH = 128
_NCH = _PPW // _CH
_WPW = _BPW // _K
_ROW = 2 * _EMB

_WE = (2 * np.arange(_PPW, dtype=np.int32)) // _K
_WO = (2 * np.arange(_PPW, dtype=np.int32) + 1) // _K

_mesh = plsc.VectorSubcoreMesh(
    core_axis_name="c", subcore_axis_name="s", num_cores=_NC, num_subcores=_NS
)


def _bucket(word, top):
    d = jnp.maximum(word - top, 1)
    one = jnp.int32(1)
    zero = jnp.int32(0)
    bl = (
        4
        + jnp.where(d >= 8, one, zero)
        + jnp.where(d >= 16, one, zero)
        + jnp.where(d >= 32, one, zero)
        + jnp.where(d >= 64, one, zero)
    )
    return jnp.where(d < 5, d - 1, bl)


def _body(tope_hbm, topo_hbm, we_hbm, wo_hbm, ptab_hbm, out_hbm,
          tope_v, topo_v, we_v, wo_v, ptab_v, rows0, rows1, osem0, osem1):
    sid = lax.axis_index("s")
    wid = sid * _NC + lax.axis_index("c")
    pbase = wid * _PPW
    pltpu.sync_copy(tope_hbm.at[pl.ds(pbase, _PPW)], tope_v)
    pltpu.sync_copy(topo_hbm.at[pl.ds(pbase, _PPW)], topo_v)
    pltpu.sync_copy(we_hbm, we_v)
    pltpu.sync_copy(wo_hbm, wo_v)
    pltpu.sync_copy(ptab_hbm, ptab_v)
    wbase = wid * _WPW
    lane = lax.iota(jnp.int32, 16)
    soff0 = lane * _ROW

    def fill(c, rowsbuf):
        def group(g, carry):
            j = c * _CH + g * 16
            te = tope_v[pl.ds(j, 16)]
            to = topo_v[pl.ds(j, 16)]
            we = we_v[pl.ds(j, 16)] + wbase
            wo = wo_v[pl.ds(j, 16)] + wbase
            be = _bucket(we, te)
            bo = _bucket(wo, to)
            gaddr = (be * 9 + bo) * _ROW
            saddr = soff0 + g * (16 * _ROW)

            @plsc.parallel_loop(0, _ROW, step=1, unroll=8)
            def colloop(col):
                v = plsc.load_gather(ptab_v, [gaddr + col])
                plsc.store_scatter(rowsbuf, [saddr + col], v)

            return carry

        lax.fori_loop(0, _CH // 16, group, 0)

    def out_ref(c):
        return out_hbm.at[pl.ds((pbase + c * _CH) * _ROW, _CH * _ROW)]

    fill(0, rows0)
    pltpu.async_copy(rows0, out_ref(0), osem0)

    def step(i, carry):
        c1 = 2 * i + 1
        @pl.when(i >= 1)
        def _():
            pltpu.make_async_copy(rows1, out_ref(c1 - 2), osem1).wait()

        fill(c1, rows1)
        pltpu.async_copy(rows1, out_ref(c1), osem1)

        @pl.when(i < _NCH // 2 - 1)
        def _():
            c2 = 2 * i + 2
            pltpu.make_async_copy(rows0, out_ref(c2 - 2), osem0).wait()
            fill(c2, rows0)
            pltpu.async_copy(rows0, out_ref(c2), osem0)

        return carry

    lax.fori_loop(0, _NCH // 2, step, 0)
    pltpu.make_async_copy(rows0, out_ref(_NCH - 2), osem0).wait()
    pltpu.make_async_copy(rows1, out_ref(_NCH - 1), osem1).wait()


_sc_lookup = pl.kernel(
    _body,
    out_type=jax.ShapeDtypeStruct((_B * _EMB,), jnp.float32),
    mesh=_mesh,
    scratch_types=[
        pltpu.VMEM((_PPW,), jnp.int32),
        pltpu.VMEM((_PPW,), jnp.int32),
        pltpu.VMEM((_PPW,), jnp.int32),
        pltpu.VMEM((_PPW,), jnp.int32),
        pltpu.VMEM((81 * _ROW,), jnp.float32),
        pltpu.VMEM((_CH * _ROW,), jnp.float32),
        pltpu.VMEM((_CH * _ROW,), jnp.float32),
        pltpu.SemaphoreType.DMA,
        pltpu.SemaphoreType.DMA,
    ],
    compiler_params=pltpu.CompilerParams(needs_layout_passes=False, disable_bounds_checks=True),
)


@jax.jit
def kernel(top_indices, distance_emb):
    emb = distance_emb.astype(jnp.float32)
    ptab = jnp.concatenate(
        [
            jnp.broadcast_to(emb[:, None, :], (9, 9, _EMB)),
            jnp.broadcast_to(emb[None, :, :], (9, 9, _EMB)),
        ],
        axis=-1,
    ).reshape(81 * _ROW)
    top_flat = top_indices.reshape(-1).astype(jnp.int32)
    tope = top_flat[0::2]
    topo = top_flat[1::2]
    out = _sc_lookup(tope, topo, jnp.asarray(_WE), jnp.asarray(_WO), ptab)
    return out.reshape(_NWORDS, _K, _EMB)


# speedup vs baseline: 1.7557x; 1.7557x over previous
import functools
import numpy as np
import jax
import jax.numpy as jnp
from jax import lax
from jax.experimental import pallas as pl
from jax.experimental.pallas import tpu as pltpu
from jax.experimental.pallas import tpu_sc as plsc

_NWORDS = 8192
_K = 50
_EMB = 64
_NC, _NS = 2, 16
_NWK = _NC * _NS
_B = _NWORDS * _K
_BPW = _B // _NWK
_PPW = _BPW // 2
_CH = 128
_NCH = _PPW // _CH
_WPW = _BPW // _K
_ROW = 2 * _EMB

_WE = (2 * np.arange(_PPW, dtype=np.int32)) // _K
_WO = (2 * np.arange(_PPW, dtype=np.int32) + 1) // _K

_mesh = plsc.VectorSubcoreMesh(
    core_axis_name="c", subcore_axis_name="s", num_cores=_NC, num_subcores=_NS
)


def _bucket(word, top):
    d = jnp.maximum(word - top, 1)
    one = jnp.int32(1)
    zero = jnp.int32(0)
    bl = (
        4
        + jnp.where(d >= 8, one, zero)
        + jnp.where(d >= 16, one, zero)
        + jnp.where(d >= 32, one, zero)
        + jnp.where(d >= 64, one, zero)
    )
    return jnp.where(d < 5, d - 1, bl)


def _body(tope_hbm, topo_hbm, we_hbm, wo_hbm, ptab_hbm, out_hbm,
          tope_v, topo_v, we_v, wo_v, ptab_v, rows0, rows1, osem0, osem1):
    sid = lax.axis_index("s")
    wid = sid * _NC + lax.axis_index("c")
    pbase = wid * _PPW
    pltpu.sync_copy(tope_hbm.at[pl.ds(pbase, _PPW)], tope_v)
    pltpu.sync_copy(topo_hbm.at[pl.ds(pbase, _PPW)], topo_v)
    pltpu.sync_copy(we_hbm, we_v)
    pltpu.sync_copy(wo_hbm, wo_v)
    pltpu.sync_copy(ptab_hbm, ptab_v)
    wbase = wid * _WPW
    lane = lax.iota(jnp.int32, 16)
    soff0 = lane * _ROW

    def fill(c, rowsbuf):
        def group(g, carry):
            j = c * _CH + g * 16
            te = tope_v[pl.ds(j, 16)]
            to = topo_v[pl.ds(j, 16)]
            we = we_v[pl.ds(j, 16)] + wbase
            wo = wo_v[pl.ds(j, 16)] + wbase
            be = _bucket(we, te)
            bo = _bucket(wo, to)
            gaddr = (be * 9 + bo) * _ROW
            saddr = soff0 + g * (16 * _ROW)

            @plsc.parallel_loop(0, _ROW, step=1, unroll=8)
            def colloop(t):
                colv = (t + lane) & (_ROW - 1)
                v = plsc.load_gather(ptab_v, [gaddr + colv])
                plsc.store_scatter(rowsbuf, [saddr + colv], v)

            return carry

        lax.fori_loop(0, _CH // 16, group, 0)

    def out_ref(c):
        return out_hbm.at[pl.ds((pbase + c * _CH) * _ROW, _CH * _ROW)]

    fill(0, rows0)
    pltpu.async_copy(rows0, out_ref(0), osem0)

    def step(i, carry):
        c1 = 2 * i + 1
        @pl.when(i >= 1)
        def _():
            pltpu.make_async_copy(rows1, out_ref(c1 - 2), osem1).wait()

        fill(c1, rows1)
        pltpu.async_copy(rows1, out_ref(c1), osem1)

        @pl.when(i < _NCH // 2 - 1)
        def _():
            c2 = 2 * i + 2
            pltpu.make_async_copy(rows0, out_ref(c2 - 2), osem0).wait()
            fill(c2, rows0)
            pltpu.async_copy(rows0, out_ref(c2), osem0)

        return carry

    lax.fori_loop(0, _NCH // 2, step, 0)
    pltpu.make_async_copy(rows0, out_ref(_NCH - 2), osem0).wait()
    pltpu.make_async_copy(rows1, out_ref(_NCH - 1), osem1).wait()


_sc_lookup = pl.kernel(
    _body,
    out_type=jax.ShapeDtypeStruct((_B * _EMB,), jnp.float32),
    mesh=_mesh,
    scratch_types=[
        pltpu.VMEM((_PPW,), jnp.int32),
        pltpu.VMEM((_PPW,), jnp.int32),
        pltpu.VMEM((_PPW,), jnp.int32),
        pltpu.VMEM((_PPW,), jnp.int32),
        pltpu.VMEM((81 * _ROW,), jnp.float32),
        pltpu.VMEM((_CH * _ROW,), jnp.float32),
        pltpu.VMEM((_CH * _ROW,), jnp.float32),
        pltpu.SemaphoreType.DMA,
        pltpu.SemaphoreType.DMA,
    ],
    compiler_params=pltpu.CompilerParams(needs_layout_passes=False, disable_bounds_checks=True),
)


@jax.jit
def kernel(top_indices, distance_emb):
    emb = distance_emb.astype(jnp.float32)
    ptab = jnp.concatenate(
        [
            jnp.broadcast_to(emb[:, None, :], (9, 9, _EMB)),
            jnp.broadcast_to(emb[None, :, :], (9, 9, _EMB)),
        ],
        axis=-1,
    ).reshape(81 * _ROW)
    top_flat = top_indices.reshape(-1).astype(jnp.int32)
    tope = top_flat[0::2]
    topo = top_flat[1::2]
    out = _sc_lookup(tope, topo, jnp.asarray(_WE), jnp.asarray(_WO), ptab)
    return out.reshape(_NWORDS, _K, _EMB)
